# Initial kernel scaffold; baseline (speedup 1.0000x reference)
import jax
import jax.numpy as jnp
from jax.experimental import pallas as pl


def kernel(input, weight):
    raise NotImplementedError("write your pallas kernel here")


# speedup vs baseline: 2.3446x; 2.3446x over previous
import functools

import jax
import jax.numpy as jnp
from jax import lax
from jax.experimental import pallas as pl
from jax.experimental.pallas import tpu as pltpu
from jax.experimental.pallas import tpu_sc as plsc

_B = 16384
_L = 50
_D = 64
_NC = 2
_NS = 16
_NW = _NC * _NS
_BAGS_W = _B // _NW
_CHUNK = 16
_NCHUNK = _BAGS_W // _CHUNK
_ROWS = _CHUNK * _L
_NSTREAM = 8
_RPS = _ROWS // _NSTREAM


def _sc_embedding_bag(idx4, weight):
    mesh = plsc.VectorSubcoreMesh(core_axis_name="c", subcore_axis_name="s")

    @functools.partial(
        pl.kernel,
        out_type=jax.ShapeDtypeStruct((_B, _D), jnp.float32),
        mesh=mesh,
        compiler_params=pltpu.CompilerParams(use_tc_tiling_on_sc=False),
        scratch_types=[
            pltpu.VMEM((_NSTREAM, _RPS), jnp.int32),
            pltpu.VMEM((_ROWS, _D), jnp.float32),
            pltpu.VMEM((_CHUNK, _D), jnp.float32),
            pltpu.SemaphoreType.DMA,
        ],
    )
    def body(idx_hbm, w_hbm, out_hbm, idx_v, rows_v, out_v, sem):
        wid = lax.axis_index("s") * _NC + lax.axis_index("c")
        bag0 = wid * _BAGS_W

        def chunk_body(g, carry):
            pltpu.sync_copy(idx_hbm.at[wid, g], idx_v)
            copies = []
            for j in range(_NSTREAM):
                c = pltpu.make_async_copy(
                    w_hbm.at[idx_v.at[j]],
                    rows_v.at[pl.ds(j * _RPS, _RPS), :],
                    sem,
                )
                c.start()
                copies.append(c)
            for c in copies:
                c.wait()

            def bag_body(i, c2):
                r0 = i * _L
                for d in range(_D // 16):
                    sl = pl.ds(d * 16, 16)
                    acc = rows_v[r0, sl]
                    for l in range(1, _L):
                        acc = acc + rows_v[r0 + l, sl]
                    out_v[i, sl] = acc * jnp.float32(1.0 / _L)
                return c2

            lax.fori_loop(0, _CHUNK, bag_body, 0)
            pltpu.sync_copy(
                out_v, out_hbm.at[pl.ds(bag0 + g * _CHUNK, _CHUNK), :]
            )
            return carry

        lax.fori_loop(0, _NCHUNK, chunk_body, 0)

    return body(idx4, weight)


def kernel(input, weight):
    idx4 = input.astype(jnp.int32).reshape(_NW, _NCHUNK, _NSTREAM, _RPS)
    return _sc_embedding_bag(idx4, weight)


# speedup vs baseline: 2.6201x; 1.1175x over previous
import functools

import jax
import jax.numpy as jnp
from jax import lax
from jax.experimental import pallas as pl
from jax.experimental.pallas import tpu as pltpu
from jax.experimental.pallas import tpu_sc as plsc

_B = 16384
_L = 50
_D = 64
_NC = 2
_NS = 16
_NW = _NC * _NS
_BAGS_W = _B // _NW
_CHUNK = 16
_NCHUNK = _BAGS_W // _CHUNK
_ROWS = _CHUNK * _L
_NSTREAM = 8
_RPS = _ROWS // _NSTREAM


def _sc_embedding_bag(idx4, weight):
    mesh = plsc.VectorSubcoreMesh(core_axis_name="c", subcore_axis_name="s")

    @functools.partial(
        pl.kernel,
        out_type=jax.ShapeDtypeStruct((_B, _D), jnp.float32),
        mesh=mesh,
        compiler_params=pltpu.CompilerParams(use_tc_tiling_on_sc=False),
        scratch_types=[
            pltpu.VMEM((_NCHUNK, _NSTREAM, _RPS), jnp.int32),
            pltpu.VMEM((2, _ROWS, _D), jnp.float32),
            pltpu.VMEM((2, _CHUNK, _D), jnp.float32),
            pltpu.SemaphoreType.DMA,
            pltpu.SemaphoreType.DMA,
            pltpu.SemaphoreType.DMA,
            pltpu.SemaphoreType.DMA,
        ],
    )
    def body(idx_hbm, w_hbm, out_hbm, idx_v, rows_v, out_v,
             gsem0, gsem1, osem0, osem1):
        gsems = (gsem0, gsem1)
        osems = (osem0, osem1)
        wid = lax.axis_index("s") * _NC + lax.axis_index("c")
        bag0 = wid * _BAGS_W

        pltpu.sync_copy(idx_hbm.at[wid], idx_v)

        def issue(g, slot):
            for j in range(_NSTREAM):
                pltpu.make_async_copy(
                    w_hbm.at[idx_v.at[g, j]],
                    rows_v.at[slot, pl.ds(j * _RPS, _RPS), :],
                    gsems[slot],
                ).start()

        def drain_gather(slot):
            pltpu.make_async_copy(
                w_hbm.at[pl.ds(0, _ROWS), :], rows_v.at[slot], gsems[slot]
            ).wait()

        def drain_out(slot):
            pltpu.make_async_copy(
                out_v.at[slot], out_hbm.at[pl.ds(0, _CHUNK), :], osems[slot]
            ).wait()

        def compute(g, slot):
            def bag_body(i, c2):
                r0 = i * _L
                for d in range(_D // 16):
                    sl = pl.ds(d * 16, 16)
                    acc = rows_v[slot, r0, sl]
                    for l in range(1, _L):
                        acc = acc + rows_v[slot, r0 + l, sl]
                    out_v[slot, i, sl] = acc * jnp.float32(1.0 / _L)
                return c2

            lax.fori_loop(0, _CHUNK, bag_body, 0)
            pltpu.make_async_copy(
                out_v.at[slot],
                out_hbm.at[pl.ds(bag0 + g * _CHUNK, _CHUNK), :],
                osems[slot],
            ).start()

        issue(0, 0)

        def pair_body(p, carry):
            for b in range(2):
                g = 2 * p + b

                @pl.when(g + 1 < _NCHUNK)
                def _():
                    issue(g + 1, 1 - b)

                drain_gather(b)

                @pl.when(g >= 2)
                def _():
                    drain_out(b)

                compute(g, b)
            return carry

        lax.fori_loop(0, _NCHUNK // 2, pair_body, 0)
        drain_out(0)
        drain_out(1)

    return body(idx4, weight)


def kernel(input, weight):
    idx4 = input.astype(jnp.int32).reshape(_NW, _NCHUNK, _NSTREAM, _RPS)
    return _sc_embedding_bag(idx4, weight)
